# Initial kernel scaffold; baseline (speedup 1.0000x reference)
import jax
import jax.numpy as jnp
from jax.experimental import pallas as pl


def kernel(x, edge_index, e, i, W0, b0, W1, b1, W2, b2, Wr, br, Wg1, bg1, Wg2, bg2, Wd1, bd1, Wd2, bd2):
    raise NotImplementedError("write your pallas kernel here")


# speedup vs baseline: 7.0764x; 7.0764x over previous
import functools

import jax
import jax.numpy as jnp
from jax import lax
from jax.experimental import pallas as pl
from jax.experimental.pallas import tpu as pltpu
from jax.experimental.pallas import tpu_sc as plsc

N = 10000
E = 160000
F = 128
DE = 4
H = 20
C1 = 16
C2 = 32
G = 64
J = H + 1
YC = J * C1
HP = 32

NC = 2
NS = 16
NW = NC * NS
EPW = E // NW
K1 = 200
NCH1 = EPW // K1
K2 = 1000
NCH2 = EPW // K2
RPS = N // NS

_mesh = plsc.VectorSubcoreMesh(
    core_axis_name="c", subcore_axis_name="s", num_cores=NC, num_subcores=NS)


def _h_body(e_ref, w0_ref, b0_ref, w1p_ref, b1p_ref, out_ref):
    h0 = jnp.maximum(
        jnp.dot(e_ref[...], w0_ref[...], preferred_element_type=jnp.float32)
        + b0_ref[...], 0.0)
    out_ref[...] = jnp.maximum(
        jnp.dot(h0, w1p_ref[...], preferred_element_type=jnp.float32)
        + b1p_ref[...], 0.0)


def _y_body(x_ref, w_ref, out_ref):
    out_ref[...] = jnp.dot(
        x_ref[...], w_ref[...], preferred_element_type=jnp.float32)


def _mid1_body(a0_ref, a1_ref, x_ref, wr_ref, br_ref,
               z1_ref, x1n2_ref, nrm_ref):
    agg = a0_ref[...] + a1_ref[...]
    deg = agg[:, C1:C1 + 1] + 1.0
    nrm = lax.rsqrt(deg)
    inv = 1.0 / deg
    x1 = jnp.maximum(
        agg[:, :C1]
        + jnp.dot(x_ref[...], wr_ref[...], preferred_element_type=jnp.float32)
        + br_ref[...], 0.0)
    z1_ref[...] = nrm * x1
    x1n2_ref[...] = inv * x1
    nrm_ref[...] = nrm


def _mid2_body(u0_ref, u1_ref, xn2_ref, nrm_ref, wg_ref, bg_ref,
               z2_ref, x2n2_ref):
    nrm = nrm_ref[...]
    p = nrm * (u0_ref[...] + u1_ref[...]) + xn2_ref[...]
    x2 = jnp.maximum(
        jnp.dot(p, wg_ref[...], preferred_element_type=jnp.float32)
        + bg_ref[...], 0.0)
    z2_ref[...] = nrm * x2
    x2n2_ref[...] = (nrm * nrm) * x2


def _tail_body(u0_ref, u1_ref, xn2_ref, nrm_ref, ids_ref, wg_ref, bg_ref,
               wd1_ref, bd1_ref, wd2_ref, bd2_ref, out_ref):
    nrm = nrm_ref[...]
    p = nrm * (u0_ref[...] + u1_ref[...]) + xn2_ref[...]
    x3 = jnp.maximum(
        jnp.dot(p, wg_ref[...], preferred_element_type=jnp.float32)
        + bg_ref[...], 0.0)
    gids = lax.broadcasted_iota(jnp.int32, (N, G), 1)
    oh = (ids_ref[...] == gids).astype(jnp.float32)
    g = lax.dot_general(oh, x3, (((0,), (0,)), ((), ())),
                        preferred_element_type=jnp.float32)
    y = jnp.dot(g, wd1_ref[...], preferred_element_type=jnp.float32) \
        + bd1_ref[...]
    out_ref[...] = jnp.dot(
        y, wd2_ref[...], preferred_element_type=jnp.float32) + bd2_ref[...]


def _sc_ecc_body(y_hbm, h_hbm, src_hbm, dst_hbm, out_hbm,
                 agg_sh, src_v, dst_v, h_v, rows_v, msg_v, zb_v, sem):
    cid = lax.axis_index("c")
    sid = lax.axis_index("s")
    wid = sid * NC + cid

    zero16 = jnp.zeros((16,), jnp.float32)

    def _zb(r, _):
        zb_v[r, pl.ds(0, 16)] = zero16
        zb_v[r, pl.ds(16, 16)] = zero16
        return 0
    lax.fori_loop(0, 25, _zb, 0)

    def _zs(t, _):
        pltpu.sync_copy(zb_v, agg_sh.at[pl.ds(sid * RPS + t * 25, 25)])
        return 0
    lax.fori_loop(0, RPS // 25, _zs, 0)

    onehot0 = jnp.where(lax.iota(jnp.int32, 16) == 0, 1.0, 0.0)

    def _mi(k, _):
        msg_v[k, pl.ds(C1, 16)] = onehot0
        return 0
    lax.fori_loop(0, K1, _mi, 0)

    pltpu.sync_copy(src_hbm.at[wid], src_v)
    pltpu.sync_copy(dst_hbm.at[wid], dst_v)

    plsc.subcore_barrier()

    def _chunk(c, _):
        base = wid * EPW + c * K1
        pltpu.sync_copy(h_hbm.at[pl.ds(base, K1)], h_v)
        pltpu.async_copy(y_hbm.at[src_v.at[c]], rows_v, sem).wait()

        def _edge(k, _):
            hv0 = h_v[k, pl.ds(0, 16)]
            hv1 = h_v[k, pl.ds(16, 16)]
            acc = zero16
            for j in range(J):
                hj = hv0[j] if j < 16 else hv1[j - 16]
                acc = acc + hj * rows_v[k, pl.ds(j * 16, 16)]
            msg_v[k, pl.ds(0, 16)] = acc
            return 0
        lax.fori_loop(0, K1, _edge, 0)

        pltpu.sync_copy(msg_v, agg_sh.at[dst_v.at[c]], add=True)
        return 0
    lax.fori_loop(0, NCH1, _chunk, 0)

    plsc.subcore_barrier()
    pltpu.sync_copy(agg_sh.at[pl.ds(sid * RPS, RPS)],
                    out_hbm.at[cid, pl.ds(sid * RPS, RPS)])


_sc_ecc = functools.partial(
    pl.kernel,
    out_type=jax.ShapeDtypeStruct((NC, N, 32), jnp.float32),
    mesh=_mesh,
    compiler_params=pltpu.CompilerParams(use_tc_tiling_on_sc=False),
    scratch_types=[
        pltpu.VMEM_SHARED((N, 32), jnp.float32),
        pltpu.VMEM((NCH1, K1), jnp.int32),
        pltpu.VMEM((NCH1, K1), jnp.int32),
        pltpu.VMEM((K1, HP), jnp.float32),
        pltpu.VMEM((K1, YC), jnp.float32),
        pltpu.VMEM((K1, 32), jnp.float32),
        pltpu.VMEM((25, 32), jnp.float32),
        pltpu.SemaphoreType.DMA,
    ],
)(_sc_ecc_body)


def _make_sc_gcn(C):
    def body(z_hbm, src_hbm, dst_hbm, out_hbm,
             u_sh, src_v, dst_v, rows_v, zb_v, sem):
        cid = lax.axis_index("c")
        sid = lax.axis_index("s")
        wid = sid * NC + cid

        zero16 = jnp.zeros((16,), jnp.float32)
        nz = C // 16

        def _zb(r, _):
            for t in range(nz):
                zb_v[r, pl.ds(t * 16, 16)] = zero16
            return 0
        lax.fori_loop(0, 25, _zb, 0)

        def _zs(t, _):
            pltpu.sync_copy(zb_v, u_sh.at[pl.ds(sid * RPS + t * 25, 25)])
            return 0
        lax.fori_loop(0, RPS // 25, _zs, 0)

        pltpu.sync_copy(src_hbm.at[wid], src_v)
        pltpu.sync_copy(dst_hbm.at[wid], dst_v)

        plsc.subcore_barrier()

        def _chunk(c, _):
            pltpu.async_copy(z_hbm.at[src_v.at[c]], rows_v, sem).wait()
            pltpu.sync_copy(rows_v, u_sh.at[dst_v.at[c]], add=True)
            return 0
        lax.fori_loop(0, NCH2, _chunk, 0)

        plsc.subcore_barrier()
        pltpu.sync_copy(u_sh.at[pl.ds(sid * RPS, RPS)],
                        out_hbm.at[cid, pl.ds(sid * RPS, RPS)])

    return functools.partial(
        pl.kernel,
        out_type=jax.ShapeDtypeStruct((NC, N, C), jnp.float32),
        mesh=_mesh,
        compiler_params=pltpu.CompilerParams(use_tc_tiling_on_sc=False),
        scratch_types=[
            pltpu.VMEM_SHARED((N, C), jnp.float32),
            pltpu.VMEM((NCH2, K2), jnp.int32),
            pltpu.VMEM((NCH2, K2), jnp.int32),
            pltpu.VMEM((K2, C), jnp.float32),
            pltpu.VMEM((25, C), jnp.float32),
            pltpu.SemaphoreType.DMA,
        ],
    )(body)


_sc_gcn16 = _make_sc_gcn(C1)
_sc_gcn32 = _make_sc_gcn(C2)


def kernel(x, edge_index, e, i, W0, b0, W1, b1, W2, b2, Wr, br,
           Wg1, bg1, Wg2, bg2, Wd1, bd1, Wd2, bd2):
    f32 = jnp.float32
    src = edge_index[0].astype(jnp.int32)
    dst = edge_index[1].astype(jnp.int32)
    src3 = src.reshape(NW, NCH1, K1)
    dst3 = dst.reshape(NW, NCH1, K1)
    src3g = src.reshape(NW, NCH2, K2)
    dst3g = dst.reshape(NW, NCH2, K2)
    ids = i.astype(jnp.int32).reshape(N, 1)

    W1p = jnp.zeros((H, HP), f32).at[:, :H].set(W1)
    b1p = jnp.zeros((HP,), f32).at[:H].set(b1).at[H].set(1.0)
    W2t = W2.reshape(H, F, C1).transpose(1, 0, 2).reshape(F, H * C1)
    W2aug = jnp.concatenate([W2t, b2.reshape(F, C1)], axis=1)

    BE = 8000
    h_aug = pl.pallas_call(
        _h_body,
        grid=(E // BE,),
        in_specs=[
            pl.BlockSpec((BE, DE), lambda m: (m, 0)),
            pl.BlockSpec((DE, H), lambda m: (0, 0)),
            pl.BlockSpec((1, H), lambda m: (0, 0)),
            pl.BlockSpec((H, HP), lambda m: (0, 0)),
            pl.BlockSpec((1, HP), lambda m: (0, 0)),
        ],
        out_specs=pl.BlockSpec((BE, HP), lambda m: (m, 0)),
        out_shape=jax.ShapeDtypeStruct((E, HP), f32),
    )(e, W0, b0.reshape(1, H), W1p, b1p.reshape(1, HP))

    BN = 1000
    Y = pl.pallas_call(
        _y_body,
        grid=(N // BN,),
        in_specs=[
            pl.BlockSpec((BN, F), lambda m: (m, 0)),
            pl.BlockSpec((F, YC), lambda m: (0, 0)),
        ],
        out_specs=pl.BlockSpec((BN, YC), lambda m: (m, 0)),
        out_shape=jax.ShapeDtypeStruct((N, YC), f32),
    )(x, W2aug)

    agg_parts = _sc_ecc(Y, h_aug, src3, dst3)

    z1, x1n2, nrm = pl.pallas_call(
        _mid1_body,
        grid=(N // BN,),
        in_specs=[
            pl.BlockSpec((BN, 32), lambda m: (m, 0)),
            pl.BlockSpec((BN, 32), lambda m: (m, 0)),
            pl.BlockSpec((BN, F), lambda m: (m, 0)),
            pl.BlockSpec((F, C1), lambda m: (0, 0)),
            pl.BlockSpec((1, C1), lambda m: (0, 0)),
        ],
        out_specs=[
            pl.BlockSpec((BN, C1), lambda m: (m, 0)),
            pl.BlockSpec((BN, C1), lambda m: (m, 0)),
            pl.BlockSpec((BN, 1), lambda m: (m, 0)),
        ],
        out_shape=[
            jax.ShapeDtypeStruct((N, C1), f32),
            jax.ShapeDtypeStruct((N, C1), f32),
            jax.ShapeDtypeStruct((N, 1), f32),
        ],
    )(agg_parts[0], agg_parts[1], x, Wr, br.reshape(1, C1))

    u1_parts = _sc_gcn16(z1, src3g, dst3g)

    z2, x2n2 = pl.pallas_call(
        _mid2_body,
        grid=(N // BN,),
        in_specs=[
            pl.BlockSpec((BN, C1), lambda m: (m, 0)),
            pl.BlockSpec((BN, C1), lambda m: (m, 0)),
            pl.BlockSpec((BN, C1), lambda m: (m, 0)),
            pl.BlockSpec((BN, 1), lambda m: (m, 0)),
            pl.BlockSpec((C1, C2), lambda m: (0, 0)),
            pl.BlockSpec((1, C2), lambda m: (0, 0)),
        ],
        out_specs=[
            pl.BlockSpec((BN, C2), lambda m: (m, 0)),
            pl.BlockSpec((BN, C2), lambda m: (m, 0)),
        ],
        out_shape=[
            jax.ShapeDtypeStruct((N, C2), f32),
            jax.ShapeDtypeStruct((N, C2), f32),
        ],
    )(u1_parts[0], u1_parts[1], x1n2, nrm, Wg1, bg1.reshape(1, C2))

    u2_parts = _sc_gcn32(z2, src3g, dst3g)

    out = pl.pallas_call(
        _tail_body,
        out_shape=jax.ShapeDtypeStruct((G, 1), f32),
    )(u2_parts[0], u2_parts[1], x2n2, nrm, ids, Wg2, bg2.reshape(1, C2),
      Wd1, bd1.reshape(1, 16), Wd2, bd2.reshape(1, 1))
    return out


# speedup vs baseline: 8.8823x; 1.2552x over previous
import functools

import jax
import jax.numpy as jnp
from jax import lax
from jax.experimental import pallas as pl
from jax.experimental.pallas import tpu as pltpu
from jax.experimental.pallas import tpu_sc as plsc

N = 10000
E = 160000
F = 128
DE = 4
H = 20
C1 = 16
C2 = 32
G = 64
J = H + 1
YC = J * C1
HP = 32

NC = 2
NS = 16
NW = NC * NS
EPW = E // NW
K1 = 40
NCH1 = EPW // K1
K2 = 1000
NCH2 = EPW // K2
RPS = N // NS

_mesh = plsc.VectorSubcoreMesh(
    core_axis_name="c", subcore_axis_name="s", num_cores=NC, num_subcores=NS)


def _h_body(e_ref, w0_ref, b0_ref, w1p_ref, b1p_ref, out_ref):
    h0 = jnp.maximum(
        jnp.dot(e_ref[...], w0_ref[...], preferred_element_type=jnp.float32)
        + b0_ref[...], 0.0)
    out_ref[...] = jnp.maximum(
        jnp.dot(h0, w1p_ref[...], preferred_element_type=jnp.float32)
        + b1p_ref[...], 0.0)


def _y_body(x_ref, w_ref, out_ref):
    out_ref[...] = jnp.dot(
        x_ref[...], w_ref[...], preferred_element_type=jnp.float32)


def _mid1_body(a0_ref, a1_ref, x_ref, wr_ref, br_ref,
               z1_ref, x1n2_ref, nrm_ref):
    agg = a0_ref[...] + a1_ref[...]
    deg = agg[:, C1:C1 + 1] + 1.0
    nrm = lax.rsqrt(deg)
    inv = 1.0 / deg
    x1 = jnp.maximum(
        agg[:, :C1]
        + jnp.dot(x_ref[...], wr_ref[...], preferred_element_type=jnp.float32)
        + br_ref[...], 0.0)
    z1_ref[...] = nrm * x1
    x1n2_ref[...] = inv * x1
    nrm_ref[...] = nrm


def _mid2_body(u0_ref, u1_ref, xn2_ref, nrm_ref, wg_ref, bg_ref,
               z2_ref, x2n2_ref):
    nrm = nrm_ref[...]
    p = nrm * (u0_ref[...] + u1_ref[...]) + xn2_ref[...]
    x2 = jnp.maximum(
        jnp.dot(p, wg_ref[...], preferred_element_type=jnp.float32)
        + bg_ref[...], 0.0)
    z2_ref[...] = nrm * x2
    x2n2_ref[...] = (nrm * nrm) * x2


def _tail_body(u0_ref, u1_ref, xn2_ref, nrm_ref, ids_ref, wg_ref, bg_ref,
               wd1_ref, bd1_ref, wd2_ref, bd2_ref, out_ref):
    nrm = nrm_ref[...]
    p = nrm * (u0_ref[...] + u1_ref[...]) + xn2_ref[...]
    x3 = jnp.maximum(
        jnp.dot(p, wg_ref[...], preferred_element_type=jnp.float32)
        + bg_ref[...], 0.0)
    gids = lax.broadcasted_iota(jnp.int32, (N, G), 1)
    oh = (ids_ref[...] == gids).astype(jnp.float32)
    g = lax.dot_general(oh, x3, (((0,), (0,)), ((), ())),
                        preferred_element_type=jnp.float32)
    y = jnp.dot(g, wd1_ref[...], preferred_element_type=jnp.float32) \
        + bd1_ref[...]
    out_ref[...] = jnp.dot(
        y, wd2_ref[...], preferred_element_type=jnp.float32) + bd2_ref[...]


def _sc_ecc_body(y_hbm, h_hbm, src_hbm, dst_hbm, out_hbm,
                 agg_sh, src_v, dst_v, h_v, rows_v, msg_v, zb_v,
                 gsem, hsem, ssem):
    cid = lax.axis_index("c")
    sid = lax.axis_index("s")
    wid = sid * NC + cid

    zero16 = jnp.zeros((16,), jnp.float32)

    def _zb(r, _):
        zb_v[r, pl.ds(0, 16)] = zero16
        zb_v[r, pl.ds(16, 16)] = zero16
        return 0
    lax.fori_loop(0, 25, _zb, 0)

    def _zs(t, _):
        pltpu.sync_copy(zb_v, agg_sh.at[pl.ds(sid * RPS + t * 25, 25)])
        return 0
    lax.fori_loop(0, RPS // 25, _zs, 0)

    onehot0 = jnp.where(lax.iota(jnp.int32, 16) == 0, 1.0, 0.0)

    def _mi(k, _):
        msg_v[0, k, pl.ds(C1, 16)] = onehot0
        msg_v[1, k, pl.ds(C1, 16)] = onehot0
        return 0
    lax.fori_loop(0, K1, _mi, 0)

    pltpu.sync_copy(src_hbm.at[wid], src_v)
    pltpu.sync_copy(dst_hbm.at[wid], dst_v)

    plsc.subcore_barrier()

    def _issue(c, par):
        base = wid * EPW + c * K1
        pltpu.async_copy(h_hbm.at[pl.ds(base, K1)], h_v.at[par],
                         hsem.at[par])
        pltpu.async_copy(y_hbm.at[src_v.at[c]], rows_v.at[par],
                         gsem.at[par])

    _issue(0, 0)

    def _chunk(c, _):
        par = lax.rem(c, 2)

        @pl.when(c + 1 < NCH1)
        def _():
            _issue(c + 1, lax.rem(c + 1, 2))

        base = wid * EPW + c * K1
        pltpu.make_async_copy(h_hbm.at[pl.ds(base, K1)], h_v.at[par],
                              hsem.at[par]).wait()
        pltpu.make_async_copy(y_hbm.at[src_v.at[c]], rows_v.at[par],
                              gsem.at[par]).wait()

        @pl.when(c >= 2)
        def _():
            pltpu.make_async_copy(msg_v.at[par],
                                  agg_sh.at[dst_v.at[c - 2]],
                                  ssem.at[par]).wait()

        def _edge(k, _):
            hv0 = h_v[par, k, pl.ds(0, 16)]
            hv1 = h_v[par, k, pl.ds(16, 16)]
            a0 = hv0[0] * rows_v[par, k, pl.ds(0, 16)]
            a1 = hv0[1] * rows_v[par, k, pl.ds(16, 16)]
            a2 = hv0[2] * rows_v[par, k, pl.ds(32, 16)]
            a3 = hv0[3] * rows_v[par, k, pl.ds(48, 16)]
            for j in range(4, J):
                hj = hv0[j] if j < 16 else hv1[j - 16]
                t = hj * rows_v[par, k, pl.ds(j * 16, 16)]
                if j % 4 == 0:
                    a0 = a0 + t
                elif j % 4 == 1:
                    a1 = a1 + t
                elif j % 4 == 2:
                    a2 = a2 + t
                else:
                    a3 = a3 + t
            msg_v[par, k, pl.ds(0, 16)] = (a0 + a1) + (a2 + a3)
            return 0
        lax.fori_loop(0, K1, _edge, 0)

        pltpu.async_copy(msg_v.at[par], agg_sh.at[dst_v.at[c]],
                         ssem.at[par], add=True)
        return 0
    lax.fori_loop(0, NCH1, _chunk, 0)

    for t in (NCH1 - 2, NCH1 - 1):
        pltpu.make_async_copy(msg_v.at[t % 2], agg_sh.at[dst_v.at[t]],
                              ssem.at[t % 2]).wait()

    plsc.subcore_barrier()
    pltpu.sync_copy(agg_sh.at[pl.ds(sid * RPS, RPS)],
                    out_hbm.at[cid, pl.ds(sid * RPS, RPS)])


_sc_ecc = functools.partial(
    pl.kernel,
    out_type=jax.ShapeDtypeStruct((NC, N, 32), jnp.float32),
    mesh=_mesh,
    compiler_params=pltpu.CompilerParams(use_tc_tiling_on_sc=False),
    scratch_types=[
        pltpu.VMEM_SHARED((N, 32), jnp.float32),
        pltpu.VMEM((NCH1, K1), jnp.int32),
        pltpu.VMEM((NCH1, K1), jnp.int32),
        pltpu.VMEM((2, K1, HP), jnp.float32),
        pltpu.VMEM((2, K1, YC), jnp.float32),
        pltpu.VMEM((2, K1, 32), jnp.float32),
        pltpu.VMEM((25, 32), jnp.float32),
        pltpu.SemaphoreType.DMA((2,)),
        pltpu.SemaphoreType.DMA((2,)),
        pltpu.SemaphoreType.DMA((2,)),
    ],
)(_sc_ecc_body)


def _make_sc_gcn(C):
    def body(z_hbm, src_hbm, dst_hbm, out_hbm,
             u_sh, src_v, dst_v, rows_v, zb_v, sem):
        cid = lax.axis_index("c")
        sid = lax.axis_index("s")
        wid = sid * NC + cid

        zero16 = jnp.zeros((16,), jnp.float32)
        nz = C // 16

        def _zb(r, _):
            for t in range(nz):
                zb_v[r, pl.ds(t * 16, 16)] = zero16
            return 0
        lax.fori_loop(0, 25, _zb, 0)

        def _zs(t, _):
            pltpu.sync_copy(zb_v, u_sh.at[pl.ds(sid * RPS + t * 25, 25)])
            return 0
        lax.fori_loop(0, RPS // 25, _zs, 0)

        pltpu.sync_copy(src_hbm.at[wid], src_v)
        pltpu.sync_copy(dst_hbm.at[wid], dst_v)

        plsc.subcore_barrier()

        def _chunk(c, _):
            pltpu.async_copy(z_hbm.at[src_v.at[c]], rows_v, sem).wait()
            pltpu.sync_copy(rows_v, u_sh.at[dst_v.at[c]], add=True)
            return 0
        lax.fori_loop(0, NCH2, _chunk, 0)

        plsc.subcore_barrier()
        pltpu.sync_copy(u_sh.at[pl.ds(sid * RPS, RPS)],
                        out_hbm.at[cid, pl.ds(sid * RPS, RPS)])

    return functools.partial(
        pl.kernel,
        out_type=jax.ShapeDtypeStruct((NC, N, C), jnp.float32),
        mesh=_mesh,
        compiler_params=pltpu.CompilerParams(use_tc_tiling_on_sc=False),
        scratch_types=[
            pltpu.VMEM_SHARED((N, C), jnp.float32),
            pltpu.VMEM((NCH2, K2), jnp.int32),
            pltpu.VMEM((NCH2, K2), jnp.int32),
            pltpu.VMEM((K2, C), jnp.float32),
            pltpu.VMEM((25, C), jnp.float32),
            pltpu.SemaphoreType.DMA,
        ],
    )(body)


_sc_gcn16 = _make_sc_gcn(C1)
_sc_gcn32 = _make_sc_gcn(C2)


def kernel(x, edge_index, e, i, W0, b0, W1, b1, W2, b2, Wr, br,
           Wg1, bg1, Wg2, bg2, Wd1, bd1, Wd2, bd2):
    f32 = jnp.float32
    src = edge_index[0].astype(jnp.int32)
    dst = edge_index[1].astype(jnp.int32)
    src3 = src.reshape(NW, NCH1, K1)
    dst3 = dst.reshape(NW, NCH1, K1)
    src3g = src.reshape(NW, NCH2, K2)
    dst3g = dst.reshape(NW, NCH2, K2)
    ids = i.astype(jnp.int32).reshape(N, 1)

    W1p = jnp.zeros((H, HP), f32).at[:, :H].set(W1)
    b1p = jnp.zeros((HP,), f32).at[:H].set(b1).at[H].set(1.0)
    W2t = W2.reshape(H, F, C1).transpose(1, 0, 2).reshape(F, H * C1)
    W2aug = jnp.concatenate([W2t, b2.reshape(F, C1)], axis=1)

    BE = 8000
    h_aug = pl.pallas_call(
        _h_body,
        grid=(E // BE,),
        in_specs=[
            pl.BlockSpec((BE, DE), lambda m: (m, 0)),
            pl.BlockSpec((DE, H), lambda m: (0, 0)),
            pl.BlockSpec((1, H), lambda m: (0, 0)),
            pl.BlockSpec((H, HP), lambda m: (0, 0)),
            pl.BlockSpec((1, HP), lambda m: (0, 0)),
        ],
        out_specs=pl.BlockSpec((BE, HP), lambda m: (m, 0)),
        out_shape=jax.ShapeDtypeStruct((E, HP), f32),
    )(e, W0, b0.reshape(1, H), W1p, b1p.reshape(1, HP))

    BN = 1000
    Y = pl.pallas_call(
        _y_body,
        grid=(N // BN,),
        in_specs=[
            pl.BlockSpec((BN, F), lambda m: (m, 0)),
            pl.BlockSpec((F, YC), lambda m: (0, 0)),
        ],
        out_specs=pl.BlockSpec((BN, YC), lambda m: (m, 0)),
        out_shape=jax.ShapeDtypeStruct((N, YC), f32),
    )(x, W2aug)

    agg_parts = _sc_ecc(Y, h_aug, src3, dst3)

    z1, x1n2, nrm = pl.pallas_call(
        _mid1_body,
        grid=(N // BN,),
        in_specs=[
            pl.BlockSpec((BN, 32), lambda m: (m, 0)),
            pl.BlockSpec((BN, 32), lambda m: (m, 0)),
            pl.BlockSpec((BN, F), lambda m: (m, 0)),
            pl.BlockSpec((F, C1), lambda m: (0, 0)),
            pl.BlockSpec((1, C1), lambda m: (0, 0)),
        ],
        out_specs=[
            pl.BlockSpec((BN, C1), lambda m: (m, 0)),
            pl.BlockSpec((BN, C1), lambda m: (m, 0)),
            pl.BlockSpec((BN, 1), lambda m: (m, 0)),
        ],
        out_shape=[
            jax.ShapeDtypeStruct((N, C1), f32),
            jax.ShapeDtypeStruct((N, C1), f32),
            jax.ShapeDtypeStruct((N, 1), f32),
        ],
    )(agg_parts[0], agg_parts[1], x, Wr, br.reshape(1, C1))

    u1_parts = _sc_gcn16(z1, src3g, dst3g)

    z2, x2n2 = pl.pallas_call(
        _mid2_body,
        grid=(N // BN,),
        in_specs=[
            pl.BlockSpec((BN, C1), lambda m: (m, 0)),
            pl.BlockSpec((BN, C1), lambda m: (m, 0)),
            pl.BlockSpec((BN, C1), lambda m: (m, 0)),
            pl.BlockSpec((BN, 1), lambda m: (m, 0)),
            pl.BlockSpec((C1, C2), lambda m: (0, 0)),
            pl.BlockSpec((1, C2), lambda m: (0, 0)),
        ],
        out_specs=[
            pl.BlockSpec((BN, C2), lambda m: (m, 0)),
            pl.BlockSpec((BN, C2), lambda m: (m, 0)),
        ],
        out_shape=[
            jax.ShapeDtypeStruct((N, C2), f32),
            jax.ShapeDtypeStruct((N, C2), f32),
        ],
    )(u1_parts[0], u1_parts[1], x1n2, nrm, Wg1, bg1.reshape(1, C2))

    u2_parts = _sc_gcn32(z2, src3g, dst3g)

    out = pl.pallas_call(
        _tail_body,
        out_shape=jax.ShapeDtypeStruct((G, 1), f32),
    )(u2_parts[0], u2_parts[1], x2n2, nrm, ids, Wg2, bg2.reshape(1, C2),
      Wd1, bd1.reshape(1, 16), Wd2, bd2.reshape(1, 1))
    return out


# speedup vs baseline: 9.4448x; 1.0633x over previous
import functools

import jax
import jax.numpy as jnp
from jax import lax
from jax.experimental import pallas as pl
from jax.experimental.pallas import tpu as pltpu
from jax.experimental.pallas import tpu_sc as plsc

N = 10000
E = 160000
F = 128
DE = 4
H = 20
C1 = 16
C2 = 32
G = 64
J = H + 1
YC = J * C1
YP = 384
HP = 32
E4 = E // 4

NC = 2
NS = 16
NW = NC * NS
EPW = E // NW
K1 = 40
NCH1 = EPW // K1
K2 = 1000
NCH2 = EPW // K2
RPS = N // NS

_mesh = plsc.VectorSubcoreMesh(
    core_axis_name="c", subcore_axis_name="s", num_cores=NC, num_subcores=NS)


def _h_body(e_ref, w0_ref, b0_ref, w1p_ref, b1p_ref, out_ref):
    h0 = jnp.maximum(
        jnp.dot(e_ref[...], w0_ref[...], preferred_element_type=jnp.float32)
        + b0_ref[...], 0.0)
    out_ref[...] = jnp.maximum(
        jnp.dot(h0, w1p_ref[...], preferred_element_type=jnp.float32)
        + b1p_ref[...], 0.0)


def _y_body(x_ref, w_ref, out_ref):
    out_ref[...] = jnp.dot(
        x_ref[...], w_ref[...], preferred_element_type=jnp.float32)


def _mid1_body(a0_ref, a1_ref, x_ref, wr_ref, br_ref,
               z1_ref, x1n2_ref, nrm_ref):
    agg = a0_ref[...] + a1_ref[...]
    deg = agg[:, C1:C1 + 1] + 1.0
    nrm = lax.rsqrt(deg)
    inv = 1.0 / deg
    x1 = jnp.maximum(
        agg[:, :C1]
        + jnp.dot(x_ref[...], wr_ref[...], preferred_element_type=jnp.float32)
        + br_ref[...], 0.0)
    z1_ref[...] = nrm * x1
    x1n2_ref[...] = inv * x1
    nrm_ref[...] = nrm


def _mid2_body(u0_ref, u1_ref, xn2_ref, nrm_ref, wg_ref, bg_ref,
               z2_ref, x2n2_ref):
    nrm = nrm_ref[...]
    p = nrm * (u0_ref[...] + u1_ref[...]) + xn2_ref[...]
    x2 = jnp.maximum(
        jnp.dot(p, wg_ref[...], preferred_element_type=jnp.float32)
        + bg_ref[...], 0.0)
    z2_ref[...] = nrm * x2
    x2n2_ref[...] = (nrm * nrm) * x2


def _tail_body(u0_ref, u1_ref, xn2_ref, nrm_ref, ids_ref, wg_ref, bg_ref,
               wd1_ref, bd1_ref, wd2_ref, bd2_ref, out_ref):
    nrm = nrm_ref[...]
    p = nrm * (u0_ref[...] + u1_ref[...]) + xn2_ref[...]
    x3 = jnp.maximum(
        jnp.dot(p, wg_ref[...], preferred_element_type=jnp.float32)
        + bg_ref[...], 0.0)
    gids = lax.broadcasted_iota(jnp.int32, (N, G), 1)
    oh = (ids_ref[...] == gids).astype(jnp.float32)
    g = lax.dot_general(oh, x3, (((0,), (0,)), ((), ())),
                        preferred_element_type=jnp.float32)
    y = jnp.dot(g, wd1_ref[...], preferred_element_type=jnp.float32) \
        + bd1_ref[...]
    out_ref[...] = jnp.dot(
        y, wd2_ref[...], preferred_element_type=jnp.float32) + bd2_ref[...]


def _sc_ecc_body(y_hbm, h_hbm, src_hbm, dst_hbm, out_hbm,
                 agg_sh, src_v, dst_v, h_v, rows_v, msg_v, zb_v,
                 gsem, hsem, ssem):
    cid = lax.axis_index("c")
    sid = lax.axis_index("s")
    wid = sid * NC + cid

    zero16 = jnp.zeros((16,), jnp.float32)

    def _zb(r, _):
        zb_v[r, pl.ds(0, 16)] = zero16
        zb_v[r, pl.ds(16, 16)] = zero16
        return 0
    lax.fori_loop(0, 25, _zb, 0)

    def _zs(t, _):
        pltpu.sync_copy(zb_v, agg_sh.at[pl.ds(sid * RPS + t * 25, 25)])
        return 0
    lax.fori_loop(0, RPS // 25, _zs, 0)

    onehot0 = jnp.where(lax.iota(jnp.int32, 16) == 0, 1.0, 0.0)

    def _mi(k, _):
        msg_v[0, k, pl.ds(C1, 16)] = onehot0
        msg_v[1, k, pl.ds(C1, 16)] = onehot0
        return 0
    lax.fori_loop(0, K1, _mi, 0)

    pltpu.sync_copy(src_hbm.at[wid], src_v)
    pltpu.sync_copy(dst_hbm.at[wid], dst_v)

    plsc.subcore_barrier()

    def _issue(c, par):
        base4 = (wid * EPW + c * K1) // 4
        pltpu.async_copy(h_hbm.at[pl.ds(base4, K1 // 4)], h_v.at[par],
                         hsem.at[par])
        pltpu.async_copy(y_hbm.at[src_v.at[c]], rows_v.at[par],
                         gsem.at[par])

    _issue(0, 0)

    def _chunk(c, _):
        par = lax.rem(c, 2)

        @pl.when(c + 1 < NCH1)
        def _():
            _issue(c + 1, lax.rem(c + 1, 2))

        base4 = (wid * EPW + c * K1) // 4
        pltpu.make_async_copy(h_hbm.at[pl.ds(base4, K1 // 4)], h_v.at[par],
                              hsem.at[par]).wait()
        pltpu.make_async_copy(y_hbm.at[src_v.at[c]], rows_v.at[par],
                              gsem.at[par]).wait()

        @pl.when(c >= 2)
        def _():
            pltpu.make_async_copy(msg_v.at[par],
                                  agg_sh.at[dst_v.at[c - 2]],
                                  ssem.at[par]).wait()

        def _edge(k, _):
            kr = lax.shift_right_logical(k, 2)
            kq = lax.mul(lax.rem(k, 4), 32)
            hv0 = h_v[par, kr, pl.ds(kq, 16)]
            hv1 = h_v[par, kr, pl.ds(kq + 16, 16)]
            a0 = hv0[0] * rows_v[par, k, pl.ds(0, 16)]
            a1 = hv0[1] * rows_v[par, k, pl.ds(16, 16)]
            a2 = hv0[2] * rows_v[par, k, pl.ds(32, 16)]
            a3 = hv0[3] * rows_v[par, k, pl.ds(48, 16)]
            for j in range(4, J):
                hj = hv0[j] if j < 16 else hv1[j - 16]
                t = hj * rows_v[par, k, pl.ds(j * 16, 16)]
                if j % 4 == 0:
                    a0 = a0 + t
                elif j % 4 == 1:
                    a1 = a1 + t
                elif j % 4 == 2:
                    a2 = a2 + t
                else:
                    a3 = a3 + t
            msg_v[par, k, pl.ds(0, 16)] = (a0 + a1) + (a2 + a3)
            return 0
        lax.fori_loop(0, K1, _edge, 0)

        pltpu.async_copy(msg_v.at[par], agg_sh.at[dst_v.at[c]],
                         ssem.at[par], add=True)
        return 0
    lax.fori_loop(0, NCH1, _chunk, 0)

    for t in (NCH1 - 2, NCH1 - 1):
        pltpu.make_async_copy(msg_v.at[t % 2], agg_sh.at[dst_v.at[t]],
                              ssem.at[t % 2]).wait()

    plsc.subcore_barrier()
    pltpu.sync_copy(agg_sh.at[pl.ds(sid * RPS, RPS)],
                    out_hbm.at[cid, pl.ds(sid * RPS, RPS)])


_sc_ecc = functools.partial(
    pl.kernel,
    out_type=jax.ShapeDtypeStruct((NC, N, 32), jnp.float32),
    mesh=_mesh,
    compiler_params=pltpu.CompilerParams(use_tc_tiling_on_sc=False),
    scratch_types=[
        pltpu.VMEM_SHARED((N, 32), jnp.float32),
        pltpu.VMEM((NCH1, K1), jnp.int32),
        pltpu.VMEM((NCH1, K1), jnp.int32),
        pltpu.VMEM((2, K1 // 4, 128), jnp.float32),
        pltpu.VMEM((2, K1, YP), jnp.float32),
        pltpu.VMEM((2, K1, 32), jnp.float32),
        pltpu.VMEM((25, 32), jnp.float32),
        pltpu.SemaphoreType.DMA((2,)),
        pltpu.SemaphoreType.DMA((2,)),
        pltpu.SemaphoreType.DMA((2,)),
    ],
)(_sc_ecc_body)


def _make_sc_gcn(C):
    def body(z_hbm, src_hbm, dst_hbm, out_hbm,
             u_sh, src_v, dst_v, rows_v, zb_v, sem):
        cid = lax.axis_index("c")
        sid = lax.axis_index("s")
        wid = sid * NC + cid

        zero16 = jnp.zeros((16,), jnp.float32)
        nz = C // 16

        def _zb(r, _):
            for t in range(nz):
                zb_v[r, pl.ds(t * 16, 16)] = zero16
            return 0
        lax.fori_loop(0, 25, _zb, 0)

        def _zs(t, _):
            pltpu.sync_copy(zb_v, u_sh.at[pl.ds(sid * RPS + t * 25, 25)])
            return 0
        lax.fori_loop(0, RPS // 25, _zs, 0)

        pltpu.sync_copy(src_hbm.at[wid], src_v)
        pltpu.sync_copy(dst_hbm.at[wid], dst_v)

        plsc.subcore_barrier()

        def _chunk(c, _):
            pltpu.async_copy(z_hbm.at[src_v.at[c]], rows_v, sem).wait()
            pltpu.sync_copy(rows_v, u_sh.at[dst_v.at[c]], add=True)
            return 0
        lax.fori_loop(0, NCH2, _chunk, 0)

        plsc.subcore_barrier()
        pltpu.sync_copy(u_sh.at[pl.ds(sid * RPS, RPS)],
                        out_hbm.at[cid, pl.ds(sid * RPS, RPS)])

    return functools.partial(
        pl.kernel,
        out_type=jax.ShapeDtypeStruct((NC, N, C), jnp.float32),
        mesh=_mesh,
        compiler_params=pltpu.CompilerParams(use_tc_tiling_on_sc=False),
        scratch_types=[
            pltpu.VMEM_SHARED((N, C), jnp.float32),
            pltpu.VMEM((NCH2, K2), jnp.int32),
            pltpu.VMEM((NCH2, K2), jnp.int32),
            pltpu.VMEM((K2, C), jnp.float32),
            pltpu.VMEM((25, C), jnp.float32),
            pltpu.SemaphoreType.DMA,
        ],
    )(body)


_sc_gcn16 = _make_sc_gcn(C1)
_sc_gcn32 = _make_sc_gcn(C2)


def kernel(x, edge_index, e, i, W0, b0, W1, b1, W2, b2, Wr, br,
           Wg1, bg1, Wg2, bg2, Wd1, bd1, Wd2, bd2):
    f32 = jnp.float32
    src = edge_index[0].astype(jnp.int32)
    dst = edge_index[1].astype(jnp.int32)
    src3 = src.reshape(NW, NCH1, K1)
    dst3 = dst.reshape(NW, NCH1, K1)
    src3g = src.reshape(NW, NCH2, K2)
    dst3g = dst.reshape(NW, NCH2, K2)
    ids = i.astype(jnp.int32).reshape(N, 1)

    W1blk = jnp.zeros((HP, HP), f32).at[:H, :H].set(W1)
    b1blk = jnp.zeros((HP,), f32).at[:H].set(b1).at[H].set(1.0)
    W0d = jnp.zeros((16, 128), f32)
    W1d = jnp.zeros((128, 128), f32)
    for q in range(4):
        W0d = W0d.at[4 * q:4 * q + DE, 32 * q:32 * q + H].set(W0)
        W1d = W1d.at[32 * q:32 * q + HP, 32 * q:32 * q + HP].set(W1blk)
    b0d = jnp.tile(jnp.zeros((HP,), f32).at[:H].set(b0), 4)
    b1d = jnp.tile(b1blk, 4)
    W2t = W2.reshape(H, F, C1).transpose(1, 0, 2).reshape(F, H * C1)
    W2aug = jnp.zeros((F, YP), f32).at[:, :H * C1].set(W2t) \
        .at[:, H * C1:YC].set(b2.reshape(F, C1))

    BE4 = 4000
    h4 = pl.pallas_call(
        _h_body,
        grid=(E4 // BE4,),
        in_specs=[
            pl.BlockSpec((BE4, 16), lambda m: (m, 0)),
            pl.BlockSpec((16, 128), lambda m: (0, 0)),
            pl.BlockSpec((1, 128), lambda m: (0, 0)),
            pl.BlockSpec((128, 128), lambda m: (0, 0)),
            pl.BlockSpec((1, 128), lambda m: (0, 0)),
        ],
        out_specs=pl.BlockSpec((BE4, 128), lambda m: (m, 0)),
        out_shape=jax.ShapeDtypeStruct((E4, 128), f32),
    )(e.reshape(E4, 16), W0d, b0d.reshape(1, 128), W1d,
      b1d.reshape(1, 128))

    BN = 1000
    Y = pl.pallas_call(
        _y_body,
        grid=(N // BN,),
        in_specs=[
            pl.BlockSpec((BN, F), lambda m: (m, 0)),
            pl.BlockSpec((F, YP), lambda m: (0, 0)),
        ],
        out_specs=pl.BlockSpec((BN, YP), lambda m: (m, 0)),
        out_shape=jax.ShapeDtypeStruct((N, YP), f32),
    )(x, W2aug)

    agg_parts = _sc_ecc(Y, h4, src3, dst3)

    z1, x1n2, nrm = pl.pallas_call(
        _mid1_body,
        grid=(N // BN,),
        in_specs=[
            pl.BlockSpec((BN, 32), lambda m: (m, 0)),
            pl.BlockSpec((BN, 32), lambda m: (m, 0)),
            pl.BlockSpec((BN, F), lambda m: (m, 0)),
            pl.BlockSpec((F, C1), lambda m: (0, 0)),
            pl.BlockSpec((1, C1), lambda m: (0, 0)),
        ],
        out_specs=[
            pl.BlockSpec((BN, C1), lambda m: (m, 0)),
            pl.BlockSpec((BN, C1), lambda m: (m, 0)),
            pl.BlockSpec((BN, 1), lambda m: (m, 0)),
        ],
        out_shape=[
            jax.ShapeDtypeStruct((N, C1), f32),
            jax.ShapeDtypeStruct((N, C1), f32),
            jax.ShapeDtypeStruct((N, 1), f32),
        ],
    )(agg_parts[0], agg_parts[1], x, Wr, br.reshape(1, C1))

    u1_parts = _sc_gcn16(z1, src3g, dst3g)

    z2, x2n2 = pl.pallas_call(
        _mid2_body,
        grid=(N // BN,),
        in_specs=[
            pl.BlockSpec((BN, C1), lambda m: (m, 0)),
            pl.BlockSpec((BN, C1), lambda m: (m, 0)),
            pl.BlockSpec((BN, C1), lambda m: (m, 0)),
            pl.BlockSpec((BN, 1), lambda m: (m, 0)),
            pl.BlockSpec((C1, C2), lambda m: (0, 0)),
            pl.BlockSpec((1, C2), lambda m: (0, 0)),
        ],
        out_specs=[
            pl.BlockSpec((BN, C2), lambda m: (m, 0)),
            pl.BlockSpec((BN, C2), lambda m: (m, 0)),
        ],
        out_shape=[
            jax.ShapeDtypeStruct((N, C2), f32),
            jax.ShapeDtypeStruct((N, C2), f32),
        ],
    )(u1_parts[0], u1_parts[1], x1n2, nrm, Wg1, bg1.reshape(1, C2))

    u2_parts = _sc_gcn32(z2, src3g, dst3g)

    out = pl.pallas_call(
        _tail_body,
        out_shape=jax.ShapeDtypeStruct((G, 1), f32),
    )(u2_parts[0], u2_parts[1], x2n2, nrm, ids, Wg2, bg2.reshape(1, C2),
      Wd1, bd1.reshape(1, 16), Wd2, bd2.reshape(1, 1))
    return out
